# Initial kernel scaffold; baseline (speedup 1.0000x reference)
import jax
import jax.numpy as jnp
from jax.experimental import pallas as pl


def kernel(wav):
    raise NotImplementedError("write your pallas kernel here")


# speedup vs baseline: 5.0325x; 5.0325x over previous
import functools

import jax
import jax.numpy as jnp
from jax import lax
from jax.experimental import pallas as pl
from jax.experimental.pallas import tpu as pltpu
from jax.experimental.pallas import tpu_sc as plsc

BATCH = 16
N_IN = 480000
N_OUT = 160000
HALF_OUT = N_OUT // 2
CHUNK_OUT = 16000
CHUNK_IN = 3 * CHUNK_OUT
NUM_CHUNKS = HALF_OUT // CHUNK_OUT
LANES = 16


def _sc_kernel(wav_hbm, out_hbm, in_v, out_v):
    nc = plsc.get_sparse_core_info().num_cores
    wid = lax.axis_index("s") * nc + lax.axis_index("c")
    row = wid // 2
    half = wid % 2
    out_base = half * HALF_OUT

    lane3 = 3 * lax.iota(jnp.int32, LANES)

    for c in range(NUM_CHUNKS):
        out_off = out_base + c * CHUNK_OUT
        in_off = 3 * out_off
        pltpu.sync_copy(wav_hbm.at[row, pl.ds(in_off, CHUNK_IN)], in_v)

        def body(j, _):
            idx = lane3 + 48 * j
            out_v[pl.ds(j * LANES, LANES)] = plsc.load_gather(in_v, [idx])
            return 0

        lax.fori_loop(0, CHUNK_OUT // LANES, body, 0)
        pltpu.sync_copy(out_v, out_hbm.at[row, pl.ds(out_off, CHUNK_OUT)])


@jax.jit
def _resample(wav):
    mesh = plsc.VectorSubcoreMesh(core_axis_name="c", subcore_axis_name="s")
    return pl.kernel(
        _sc_kernel,
        mesh=mesh,
        out_type=jax.ShapeDtypeStruct((BATCH, N_OUT), jnp.float32),
        scratch_types=[
            pltpu.VMEM((CHUNK_IN,), jnp.float32),
            pltpu.VMEM((CHUNK_OUT,), jnp.float32),
        ],
        compiler_params=pltpu.CompilerParams(needs_layout_passes=False),
    )(wav)


def kernel(wav):
    wav = wav.reshape(wav.shape[0], -1)
    return _resample(wav)
